# baseline (speedup 1.0000x reference)
import jax
import jax.numpy as jnp
from jax.experimental import pallas as pl
from jax.experimental.pallas import tpu as pltpu

_IMG = 224
_NPIX = _IMG * _IMG
_P = 12544
_NB = _NPIX // _P
_SW = 1792
_NS = _P // _SW
_HB = 256
_WW = _SW + 2 * _HB
_HPB = _P // _HB
_NHB = _NPIX // _HB


def _conv_block(w_ref, b_ref, m_ref, xl_ref, xm_ref, xr_ref, o_ref):
    i = pl.program_id(1)
    fl = jnp.where(i == 0, 0, 1).astype(jnp.bfloat16)
    fr = jnp.where(i == _NB - 1, 0, 1).astype(jnp.bfloat16)
    for s in range(_NS):
        lo = s * _SW - _HB
        hi = s * _SW + _SW + _HB
        if s == 0:
            ws = jnp.concatenate(
                [xl_ref[0].astype(jnp.bfloat16) * fl,
                 xm_ref[0, :, 0:hi].astype(jnp.bfloat16)], axis=1)
        elif s == _NS - 1:
            ws = jnp.concatenate(
                [xm_ref[0, :, lo:_P].astype(jnp.bfloat16),
                 xr_ref[0].astype(jnp.bfloat16) * fr], axis=1)
        else:
            ws = xm_ref[0, :, lo:hi].astype(jnp.bfloat16)
        acc = jnp.zeros((o_ref.shape[1], _SW), jnp.float32)
        for kh in range(3):
            for kw in range(3):
                t = kh * 3 + kw
                o = _HB + (kh - 1) * _IMG + (kw - 1)
                bt = ws[:, o:o + _SW]
                if kw != 1:
                    bt = bt * m_ref[kw // 2:kw // 2 + 1, o:o + _SW]
                acc += jax.lax.dot_general(
                    w_ref[t], bt,
                    dimension_numbers=(((1,), (0,)), ((), ())),
                    preferred_element_type=jnp.float32,
                )
        o_ref[0, :, s * _SW:(s + 1) * _SW] = acc + b_ref[:]


def kernel(x, kernel, bias):
    batch, cin, img, _ = x.shape
    cout = kernel.shape[0]
    wt = kernel.transpose(2, 3, 0, 1).reshape(9, cout, cin).astype(jnp.bfloat16)
    b2 = bias.reshape(cout, 1)
    x3 = x.reshape(batch, cin, _NPIX)

    l = jnp.arange(_WW, dtype=jnp.int32)
    col = (l + (_IMG - _HB % _IMG)) % _IMG
    masks = jnp.stack([(col != _IMG - 1), (col != 0)]).astype(jnp.bfloat16)

    out_flat = pl.pallas_call(
        _conv_block,
        grid=(batch, _NB),
        in_specs=[
            pl.BlockSpec((9, cout, cin), lambda b, i: (0, 0, 0)),
            pl.BlockSpec((cout, 1), lambda b, i: (0, 0)),
            pl.BlockSpec((2, _WW), lambda b, i: (0, 0)),
            pl.BlockSpec((1, cin, _HB),
                         lambda b, i: (b, 0, jnp.maximum(i * _HPB - 1, 0))),
            pl.BlockSpec((1, cin, _P), lambda b, i: (b, 0, i)),
            pl.BlockSpec((1, cin, _HB),
                         lambda b, i: (b, 0, jnp.minimum(i * _HPB + _HPB,
                                                         _NHB - 1))),
        ],
        out_specs=pl.BlockSpec((1, cout, _P), lambda b, i: (b, 0, i)),
        out_shape=jax.ShapeDtypeStruct((batch, cout, _NPIX), jnp.float32),
        compiler_params=pltpu.CompilerParams(
            dimension_semantics=("parallel", "parallel")),
    )(wt, b2, masks, x3, x3, x3)

    return out_flat.reshape(batch, cout, img, img)


# speedup vs baseline: 1.0009x; 1.0009x over previous
import jax
import jax.numpy as jnp
from jax.experimental import pallas as pl
from jax.experimental.pallas import tpu as pltpu

_IMG = 224
_NPIX = _IMG * _IMG
_P = 25088
_NB = _NPIX // _P
_SW = 1792
_NS = _P // _SW
_HB = 256
_WW = _SW + 2 * _HB
_HPB = _P // _HB
_NHB = _NPIX // _HB


def _conv_block(w_ref, b_ref, m_ref, xl_ref, xm_ref, xr_ref, o_ref):
    i = pl.program_id(1)
    fl = jnp.where(i == 0, 0, 1).astype(jnp.bfloat16)
    fr = jnp.where(i == _NB - 1, 0, 1).astype(jnp.bfloat16)
    for s in range(_NS):
        lo = s * _SW - _HB
        hi = s * _SW + _SW + _HB
        if s == 0:
            ws = jnp.concatenate(
                [xl_ref[0].astype(jnp.bfloat16) * fl,
                 xm_ref[0, :, 0:hi].astype(jnp.bfloat16)], axis=1)
        elif s == _NS - 1:
            ws = jnp.concatenate(
                [xm_ref[0, :, lo:_P].astype(jnp.bfloat16),
                 xr_ref[0].astype(jnp.bfloat16) * fr], axis=1)
        else:
            ws = xm_ref[0, :, lo:hi].astype(jnp.bfloat16)
        acc = jnp.zeros((o_ref.shape[1], _SW), jnp.float32)
        for kh in range(3):
            for kw in range(3):
                t = kh * 3 + kw
                o = _HB + (kh - 1) * _IMG + (kw - 1)
                bt = ws[:, o:o + _SW]
                if kw != 1:
                    bt = bt * m_ref[kw // 2:kw // 2 + 1, o:o + _SW]
                acc += jax.lax.dot_general(
                    w_ref[t], bt,
                    dimension_numbers=(((1,), (0,)), ((), ())),
                    preferred_element_type=jnp.float32,
                )
        o_ref[0, :, s * _SW:(s + 1) * _SW] = acc + b_ref[:]


def kernel(x, kernel, bias):
    batch, cin, img, _ = x.shape
    cout = kernel.shape[0]
    wt = kernel.transpose(2, 3, 0, 1).reshape(9, cout, cin).astype(jnp.bfloat16)
    b2 = bias.reshape(cout, 1)
    x3 = x.reshape(batch, cin, _NPIX)

    l = jnp.arange(_WW, dtype=jnp.int32)
    col = (l + (_IMG - _HB % _IMG)) % _IMG
    masks = jnp.stack([(col != _IMG - 1), (col != 0)]).astype(jnp.bfloat16)

    out_flat = pl.pallas_call(
        _conv_block,
        grid=(batch, _NB),
        in_specs=[
            pl.BlockSpec((9, cout, cin), lambda b, i: (0, 0, 0)),
            pl.BlockSpec((cout, 1), lambda b, i: (0, 0)),
            pl.BlockSpec((2, _WW), lambda b, i: (0, 0)),
            pl.BlockSpec((1, cin, _HB),
                         lambda b, i: (b, 0, jnp.maximum(i * _HPB - 1, 0))),
            pl.BlockSpec((1, cin, _P), lambda b, i: (b, 0, i)),
            pl.BlockSpec((1, cin, _HB),
                         lambda b, i: (b, 0, jnp.minimum(i * _HPB + _HPB,
                                                         _NHB - 1))),
        ],
        out_specs=pl.BlockSpec((1, cout, _P), lambda b, i: (b, 0, i)),
        out_shape=jax.ShapeDtypeStruct((batch, cout, _NPIX), jnp.float32),
        compiler_params=pltpu.CompilerParams(
            dimension_semantics=("parallel", "parallel")),
    )(wt, b2, masks, x3, x3, x3)

    return out_flat.reshape(batch, cout, img, img)


# speedup vs baseline: 1.0023x; 1.0014x over previous
import jax
import jax.numpy as jnp
from jax.experimental import pallas as pl
from jax.experimental.pallas import tpu as pltpu

_IMG = 224
_NPIX = _IMG * _IMG
_P = 25088
_NB = _NPIX // _P
_SW = 3584
_NS = _P // _SW
_HB = 256
_WW = _SW + 2 * _HB
_HPB = _P // _HB
_NHB = _NPIX // _HB


def _conv_block(w_ref, b_ref, m_ref, xl_ref, xm_ref, xr_ref, o_ref):
    i = pl.program_id(1)
    fl = jnp.where(i == 0, 0, 1).astype(jnp.bfloat16)
    fr = jnp.where(i == _NB - 1, 0, 1).astype(jnp.bfloat16)
    for s in range(_NS):
        lo = s * _SW - _HB
        hi = s * _SW + _SW + _HB
        if s == 0:
            ws = jnp.concatenate(
                [xl_ref[0].astype(jnp.bfloat16) * fl,
                 xm_ref[0, :, 0:hi].astype(jnp.bfloat16)], axis=1)
        elif s == _NS - 1:
            ws = jnp.concatenate(
                [xm_ref[0, :, lo:_P].astype(jnp.bfloat16),
                 xr_ref[0].astype(jnp.bfloat16) * fr], axis=1)
        else:
            ws = xm_ref[0, :, lo:hi].astype(jnp.bfloat16)
        acc = jnp.zeros((o_ref.shape[1], _SW), jnp.float32)
        for kh in range(3):
            for kw in range(3):
                t = kh * 3 + kw
                o = _HB + (kh - 1) * _IMG + (kw - 1)
                bt = ws[:, o:o + _SW]
                if kw != 1:
                    bt = bt * m_ref[kw // 2:kw // 2 + 1, o:o + _SW]
                acc += jax.lax.dot_general(
                    w_ref[t], bt,
                    dimension_numbers=(((1,), (0,)), ((), ())),
                    preferred_element_type=jnp.float32,
                )
        o_ref[0, :, s * _SW:(s + 1) * _SW] = acc + b_ref[:]


def kernel(x, kernel, bias):
    batch, cin, img, _ = x.shape
    cout = kernel.shape[0]
    wt = kernel.transpose(2, 3, 0, 1).reshape(9, cout, cin).astype(jnp.bfloat16)
    b2 = bias.reshape(cout, 1)
    x3 = x.reshape(batch, cin, _NPIX)

    l = jnp.arange(_WW, dtype=jnp.int32)
    col = (l + (_IMG - _HB % _IMG)) % _IMG
    masks = jnp.stack([(col != _IMG - 1), (col != 0)]).astype(jnp.bfloat16)

    out_flat = pl.pallas_call(
        _conv_block,
        grid=(batch, _NB),
        in_specs=[
            pl.BlockSpec((9, cout, cin), lambda b, i: (0, 0, 0)),
            pl.BlockSpec((cout, 1), lambda b, i: (0, 0)),
            pl.BlockSpec((2, _WW), lambda b, i: (0, 0)),
            pl.BlockSpec((1, cin, _HB),
                         lambda b, i: (b, 0, jnp.maximum(i * _HPB - 1, 0))),
            pl.BlockSpec((1, cin, _P), lambda b, i: (b, 0, i)),
            pl.BlockSpec((1, cin, _HB),
                         lambda b, i: (b, 0, jnp.minimum(i * _HPB + _HPB,
                                                         _NHB - 1))),
        ],
        out_specs=pl.BlockSpec((1, cout, _P), lambda b, i: (b, 0, i)),
        out_shape=jax.ShapeDtypeStruct((batch, cout, _NPIX), jnp.float32),
        compiler_params=pltpu.CompilerParams(
            dimension_semantics=("parallel", "parallel")),
    )(wt, b2, masks, x3, x3, x3)

    return out_flat.reshape(batch, cout, img, img)


# speedup vs baseline: 1.0067x; 1.0043x over previous
import jax
import jax.numpy as jnp
from jax.experimental import pallas as pl
from jax.experimental.pallas import tpu as pltpu

_IMG = 224
_NPIX = _IMG * _IMG
_P = 12544
_NB = _NPIX // _P
_HB = 256
_HPB = _P // _HB
_NHB = _NPIX // _HB
_CW = _P + 2 * _HB


def _conv_block(w_ref, b_ref, m_ref, xl_ref, xm_ref, xr_ref, o_ref):
    i = pl.program_id(1)
    fl = jnp.where(i == 0, 0, 1).astype(jnp.bfloat16)
    fr = jnp.where(i == _NB - 1, 0, 1).astype(jnp.bfloat16)
    cb1 = jnp.concatenate(
        [xl_ref[0].astype(jnp.bfloat16) * fl,
         xm_ref[0].astype(jnp.bfloat16),
         xr_ref[0].astype(jnp.bfloat16) * fr], axis=1)
    cbs = [cb1 * m_ref[0:1, :], cb1, cb1 * m_ref[1:2, :]]
    acc = jnp.zeros((o_ref.shape[1], _P), jnp.float32)
    for kh in range(3):
        for kw in range(3):
            t = kh * 3 + kw
            o = _HB + (kh - 1) * _IMG + (kw - 1)
            acc += jax.lax.dot_general(
                w_ref[t], cbs[kw][:, o:o + _P],
                dimension_numbers=(((1,), (0,)), ((), ())),
                preferred_element_type=jnp.float32,
            )
    o_ref[0] = acc + b_ref[:]


def kernel(x, kernel, bias):
    batch, cin, img, _ = x.shape
    cout = kernel.shape[0]
    wt = kernel.transpose(2, 3, 0, 1).reshape(9, cout, cin).astype(jnp.bfloat16)
    b2 = bias.reshape(cout, 1)
    x3 = x.reshape(batch, cin, _NPIX)

    l = jnp.arange(_CW, dtype=jnp.int32)
    col = (l + 192) % _IMG
    masks = jnp.stack([(col != _IMG - 1), (col != 0)]).astype(jnp.bfloat16)

    out_flat = pl.pallas_call(
        _conv_block,
        grid=(batch, _NB),
        in_specs=[
            pl.BlockSpec((9, cout, cin), lambda b, i: (0, 0, 0)),
            pl.BlockSpec((cout, 1), lambda b, i: (0, 0)),
            pl.BlockSpec((2, _CW), lambda b, i: (0, 0)),
            pl.BlockSpec((1, cin, _HB),
                         lambda b, i: (b, 0, jnp.maximum(i * _HPB - 1, 0))),
            pl.BlockSpec((1, cin, _P), lambda b, i: (b, 0, i)),
            pl.BlockSpec((1, cin, _HB),
                         lambda b, i: (b, 0, jnp.minimum(i * _HPB + _HPB,
                                                         _NHB - 1))),
        ],
        out_specs=pl.BlockSpec((1, cout, _P), lambda b, i: (b, 0, i)),
        out_shape=jax.ShapeDtypeStruct((batch, cout, _NPIX), jnp.float32),
        compiler_params=pltpu.CompilerParams(
            dimension_semantics=("parallel", "parallel")),
    )(wt, b2, masks, x3, x3, x3)

    return out_flat.reshape(batch, cout, img, img)
